# baseline (speedup 1.0000x reference)
import functools

import jax
import jax.numpy as jnp
from jax import lax
from jax.experimental import pallas as pl
from jax.experimental.pallas import tpu as pltpu
from jax.experimental.pallas import tpu_sc as plsc

N = 320000
C = 10
M = 16
G = 8
NUM_GRAPHS = 512

NC = 2
NS = 16
NW = NC * NS
CHUNK = N // NW
SUB = 128
J_FULL = CHUNK // SUB
REM = CHUNK - J_FULL * SUB
WIDE = 6
L = 16


def _tables_body(b_ref, pi_ref, out_ref):
    pi = pi_ref[...]
    pi = pi - jnp.max(pi, axis=0, keepdims=True)
    epi = jnp.exp(pi)
    smpi = epi / jnp.sum(epi, axis=0, keepdims=True)

    nums = []
    denom = jnp.zeros((M, G), jnp.float32)
    for c in range(C):
        bc = b_ref[c]
        bc = bc - jnp.max(bc, axis=0, keepdims=True)
        eb = jnp.exp(bc)
        smb = eb / jnp.sum(eb, axis=0, keepdims=True)
        num = smpi[c:c + 1, :] * smb
        nums.append(num)
        denom = denom + num

    lik = jnp.zeros((M, G), jnp.float32)
    best = jnp.full((M, G), -jnp.inf, jnp.float32)
    best_idx = jnp.zeros((M, G), jnp.int32)
    for c in range(C):
        post = nums[c] / denom
        lik = lik + post * jnp.log(nums[c])
        upd = nums[c] > best
        best_idx = jnp.where(upd, jnp.int32(c), best_idx)
        best = jnp.where(upd, nums[c], best)

    comb = jnp.concatenate(
        [lik, lax.bitcast_convert_type(best_idx, jnp.float32)], axis=1)
    out_ref[...] = comb


def _tables(B, Pi):
    return pl.pallas_call(
        _tables_body,
        out_shape=jax.ShapeDtypeStruct((M, 2 * G), jnp.float32),
    )(B, Pi)


def _combine_body(p_ref, out_ref):
    out_ref[...] = p_ref[0] + p_ref[1]


def _combine(parts):
    return pl.pallas_call(
        _combine_body,
        out_shape=jax.ShapeDtypeStruct((NUM_GRAPHS, G), jnp.float32),
    )(parts)


def _build_slice(tab, x_ref, hbuf, likbuf, iota):
    for v in range(SUB // L):
        xv = x_ref[pl.ds(v * L, L)]
        xb = xv * (2 * G)
        hrows = jnp.full((L,), v, jnp.int32)
        lrows = iota + v * L
        for j in range(G):
            lval = plsc.load_gather(tab, [xb + j])
            hval = plsc.load_gather(tab, [xb + (G + j)])
            cols = jnp.full((L,), j, jnp.int32)
            plsc.store_scatter(likbuf, [lrows, cols], lval)
            plsc.store_scatter(hbuf, [hrows, iota * G + j],
                               plsc.bitcast(hval, jnp.int32))


def _sc_body(x_hbm, batch_hbm, tab_hbm, zeros_hbm,
             hout_hbm, likp_hbm,
             tab, x_s, b_s, hbuf, lbuf,
             x_r, b_r, hbuf_r, lbuf_r,
             acc, sem_l, sem_w, sem_a, sem_r):
    cid = lax.axis_index("c")
    sid = lax.axis_index("s")
    wid = cid * NS + sid
    base = wid * CHUNK
    iota = lax.iota(jnp.int32, L)

    @pl.when(sid == 0)
    def _():
        pltpu.sync_copy(zeros_hbm, acc)

    pltpu.sync_copy(tab_hbm, tab)

    plsc.subcore_barrier()

    def outer(i, carry):
        lcps = []
        for b in range(WIDE):
            off = pl.multiple_of(base + (WIDE * i + b) * SUB, 8)
            lcps.append(
                (pltpu.async_copy(x_hbm.at[pl.ds(off, SUB)], x_s[b], sem_l),
                 pltpu.async_copy(batch_hbm.at[pl.ds(off, SUB)], b_s[b],
                                  sem_l)))
        for a, bb in lcps:
            a.wait()
            bb.wait()
        for b in range(WIDE):
            _build_slice(tab, x_s[b], hbuf[b], lbuf[b], iota)
        wcps = []
        acps = []
        for b in range(WIDE):
            off = pl.multiple_of(base + (WIDE * i + b) * SUB, 8)
            hrow = off // L
            wcps.append(
                pltpu.async_copy(hbuf[b],
                                 hout_hbm.at[pl.ds(hrow, SUB * G // 128)],
                                 sem_w))
            acps.append(
                pltpu.async_copy(lbuf[b], acc.at[b_s[b]], sem_a, add=True))
        for cp in wcps:
            cp.wait()
        for cp in acps:
            cp.wait()
        return carry

    lax.fori_loop(0, J_FULL // WIDE, outer, 0)

    off_r = base + J_FULL * SUB
    pltpu.sync_copy(x_hbm.at[pl.ds(off_r, REM)], x_r)
    pltpu.sync_copy(batch_hbm.at[pl.ds(off_r, REM)], b_r)
    xv = x_r[...]
    xb = xv * (2 * G)
    for j in range(G):
        lval = plsc.load_gather(tab, [xb + j])
        hval = plsc.load_gather(tab, [xb + (G + j)])
        cols = jnp.full((L,), j, jnp.int32)
        plsc.store_scatter(lbuf_r, [iota, cols], lval)
        plsc.store_scatter(hbuf_r, [jnp.full((L,), 0, jnp.int32),
                                    iota * G + j],
                           plsc.bitcast(hval, jnp.int32))
    pltpu.sync_copy(hbuf_r, hout_hbm.at[pl.ds(off_r // L, REM * G // 128)])
    pltpu.sync_copy(lbuf_r, acc.at[b_r], add=True)

    plsc.subcore_barrier()

    @pl.when(sid == 0)
    def _():
        pltpu.sync_copy(acc, likp_hbm.at[cid])


@functools.lru_cache(maxsize=1)
def _sc_main():
    mesh = plsc.VectorSubcoreMesh(
        core_axis_name="c", subcore_axis_name="s",
        num_cores=NC, num_subcores=NS)
    return pl.kernel(
        _sc_body,
        out_type=(
            jax.ShapeDtypeStruct((N * G // 128, 128), jnp.int32),
            jax.ShapeDtypeStruct((NC, NUM_GRAPHS, G), jnp.float32),
        ),
        mesh=mesh,
        scratch_types=[
            pltpu.VMEM((2 * M * G,), jnp.float32),
            [pltpu.VMEM((SUB,), jnp.int32)] * WIDE,
            [pltpu.VMEM((SUB,), jnp.int32)] * WIDE,
            [pltpu.VMEM((SUB * G // 128, 128), jnp.int32)] * WIDE,
            [pltpu.VMEM((SUB, G), jnp.float32)] * WIDE,
            pltpu.VMEM((REM,), jnp.int32),
            pltpu.VMEM((REM,), jnp.int32),
            pltpu.VMEM((REM * G // 128, 128), jnp.int32),
            pltpu.VMEM((REM, G), jnp.float32),
            pltpu.VMEM_SHARED((NUM_GRAPHS, G), jnp.float32),
            pltpu.SemaphoreType.DMA,
            pltpu.SemaphoreType.DMA,
            pltpu.SemaphoreType.DMA,
            pltpu.SemaphoreType.DMA,
        ],
        compiler_params=pltpu.CompilerParams(
            use_tc_tiling_on_sc=False, needs_layout_passes=False),
    )


def kernel(x, batch, B, Pi):
    tab = _tables(B.astype(jnp.float32), Pi.astype(jnp.float32))
    tab = tab.reshape(2 * M * G)
    zeros = jnp.zeros((NUM_GRAPHS, G), jnp.float32)
    h_wide, lik_part = _sc_main()(
        x.astype(jnp.int32), batch.astype(jnp.int32), tab, zeros)
    likelihood = _combine(lik_part)
    return likelihood, h_wide.reshape(N, G)


# speedup vs baseline: 1.0464x; 1.0464x over previous
import functools

import jax
import jax.numpy as jnp
from jax import lax
from jax.experimental import pallas as pl
from jax.experimental.pallas import tpu as pltpu
from jax.experimental.pallas import tpu_sc as plsc

N = 320000
C = 10
M = 16
G = 8
NUM_GRAPHS = 512

NC = 2
NS = 16
NW = NC * NS
CHUNK = N // NW
SUB = 128
J_FULL = CHUNK // SUB
REM = CHUNK - J_FULL * SUB
WIDE = 6
L = 16


def _tables_body(b_ref, pi_ref, out_ref):
    pi = pi_ref[...]
    pi = pi - jnp.max(pi, axis=0, keepdims=True)
    epi = jnp.exp(pi)
    smpi = epi / jnp.sum(epi, axis=0, keepdims=True)

    nums = []
    denom = jnp.zeros((M, G), jnp.float32)
    for c in range(C):
        bc = b_ref[c]
        bc = bc - jnp.max(bc, axis=0, keepdims=True)
        eb = jnp.exp(bc)
        smb = eb / jnp.sum(eb, axis=0, keepdims=True)
        num = smpi[c:c + 1, :] * smb
        nums.append(num)
        denom = denom + num

    lik = jnp.zeros((M, G), jnp.float32)
    best = jnp.full((M, G), -jnp.inf, jnp.float32)
    best_idx = jnp.zeros((M, G), jnp.int32)
    for c in range(C):
        post = nums[c] / denom
        lik = lik + post * jnp.log(nums[c])
        upd = nums[c] > best
        best_idx = jnp.where(upd, jnp.int32(c), best_idx)
        best = jnp.where(upd, nums[c], best)

    comb = jnp.concatenate(
        [lik, lax.bitcast_convert_type(best_idx, jnp.float32)], axis=1)
    out_ref[...] = comb


def _tables(B, Pi):
    return pl.pallas_call(
        _tables_body,
        out_shape=jax.ShapeDtypeStruct((M, 2 * G), jnp.float32),
    )(B, Pi)


def _combine_body(p_ref, out_ref):
    out_ref[...] = p_ref[0] + p_ref[1]


def _combine(parts):
    return pl.pallas_call(
        _combine_body,
        out_shape=jax.ShapeDtypeStruct((NUM_GRAPHS, G), jnp.float32),
    )(parts)


def _build_slice(tab, x_ref, hbuf, likbuf, iota):
    def body(v, carry):
        xv = x_ref[pl.ds(v * L, L)]
        xb = xv * (2 * G)
        for j in range(G):
            lval = plsc.load_gather(tab, [xb + j])
            hval = plsc.load_gather(tab, [xb + (G + j)])
            rows = iota + v * L
            cols = jnp.full((L,), j, jnp.int32)
            plsc.store_scatter(likbuf, [rows, cols], lval)
            plsc.store_scatter(
                hbuf, [iota * G + (v * L * G + j)],
                plsc.bitcast(hval, jnp.int32))
        return carry
    lax.fori_loop(0, SUB // L, body, 0)


def _sc_body(x_hbm, batch_hbm, tab_hbm,
             hout_hbm, likp_hbm,
             tab, x_s, b_s, hbuf, lbuf, zbuf,
             x_r, b_r, hbuf_r, lbuf_r,
             acc, sem_l, sem_w, sem_a, sem_r):
    cid = lax.axis_index("c")
    sid = lax.axis_index("s")
    wid = cid * NS + sid
    base = wid * CHUNK
    iota = lax.iota(jnp.int32, L)

    @pl.when(sid == 0)
    def _():
        zero16 = jnp.zeros((L,), jnp.float32)
        for v in range(SUB // L):
            rows = iota + v * L
            for j in range(G):
                plsc.store_scatter(
                    zbuf, [rows, jnp.full((L,), j, jnp.int32)], zero16)
        for k in range(NUM_GRAPHS // SUB):
            pltpu.sync_copy(zbuf, acc.at[pl.ds(k * SUB, SUB)])

    pltpu.sync_copy(tab_hbm, tab)

    plsc.subcore_barrier()

    def drain_writes():
        for _ in range(WIDE):
            pltpu.make_async_copy(
                hbuf[0], hout_hbm.at[pl.ds(0, SUB * G)], sem_w).wait()
            pltpu.make_async_copy(
                lbuf[0], acc.at[b_s[0]], sem_a).wait()

    def outer(i, carry):
        @pl.when(i > 0)
        def _():
            drain_writes()
        lcps = []
        for b in range(WIDE):
            off = pl.multiple_of(base + (WIDE * i + b) * SUB, 8)
            lcps.append(
                (pltpu.async_copy(x_hbm.at[pl.ds(off, SUB)], x_s[b], sem_l),
                 pltpu.async_copy(batch_hbm.at[pl.ds(off, SUB)], b_s[b],
                                  sem_l)))
        for a, bb in lcps:
            a.wait()
            bb.wait()
        for b in range(WIDE):
            _build_slice(tab, x_s[b], hbuf[b], lbuf[b], iota)
        for b in range(WIDE):
            off = pl.multiple_of(base + (WIDE * i + b) * SUB, 8)
            hoff = pl.multiple_of(off * G, 8)
            pltpu.async_copy(hbuf[b], hout_hbm.at[pl.ds(hoff, SUB * G)],
                             sem_w)
            pltpu.async_copy(lbuf[b], acc.at[b_s[b]], sem_a, add=True)
        return carry

    lax.fori_loop(0, J_FULL // WIDE, outer, 0)
    drain_writes()

    off_r = base + J_FULL * SUB
    pltpu.sync_copy(x_hbm.at[pl.ds(off_r, REM)], x_r)
    pltpu.sync_copy(batch_hbm.at[pl.ds(off_r, REM)], b_r)
    xv = x_r[...]
    xb = xv * (2 * G)
    for j in range(G):
        lval = plsc.load_gather(tab, [xb + j])
        hval = plsc.load_gather(tab, [xb + (G + j)])
        cols = jnp.full((L,), j, jnp.int32)
        plsc.store_scatter(lbuf_r, [iota, cols], lval)
        plsc.store_scatter(
            hbuf_r, [iota * G + j], plsc.bitcast(hval, jnp.int32))
    pltpu.sync_copy(hbuf_r, hout_hbm.at[pl.ds(off_r * G, REM * G)])
    pltpu.sync_copy(lbuf_r, acc.at[b_r], add=True)

    plsc.subcore_barrier()

    @pl.when(sid == 0)
    def _():
        pltpu.sync_copy(acc, likp_hbm.at[cid])


@functools.lru_cache(maxsize=1)
def _sc_main():
    mesh = plsc.VectorSubcoreMesh(
        core_axis_name="c", subcore_axis_name="s",
        num_cores=NC, num_subcores=NS)
    return pl.kernel(
        _sc_body,
        out_type=(
            jax.ShapeDtypeStruct((N * G,), jnp.int32),
            jax.ShapeDtypeStruct((NC, NUM_GRAPHS, G), jnp.float32),
        ),
        mesh=mesh,
        scratch_types=[
            pltpu.VMEM((2 * M * G,), jnp.float32),
            [pltpu.VMEM((SUB,), jnp.int32)] * WIDE,
            [pltpu.VMEM((SUB,), jnp.int32)] * WIDE,
            [pltpu.VMEM((SUB * G,), jnp.int32)] * WIDE,
            [pltpu.VMEM((SUB, G), jnp.float32)] * WIDE,
            pltpu.VMEM((SUB, G), jnp.float32),
            pltpu.VMEM((REM,), jnp.int32),
            pltpu.VMEM((REM,), jnp.int32),
            pltpu.VMEM((REM * G,), jnp.int32),
            pltpu.VMEM((REM, G), jnp.float32),
            pltpu.VMEM_SHARED((NUM_GRAPHS, G), jnp.float32),
            pltpu.SemaphoreType.DMA,
            pltpu.SemaphoreType.DMA,
            pltpu.SemaphoreType.DMA,
            pltpu.SemaphoreType.DMA,
        ],
        compiler_params=pltpu.CompilerParams(
            use_tc_tiling_on_sc=False, needs_layout_passes=False),
    )


def kernel(x, batch, B, Pi):
    tab = _tables(B.astype(jnp.float32), Pi.astype(jnp.float32))
    tab = tab.reshape(2 * M * G)
    h_flat, lik_part = _sc_main()(
        x.astype(jnp.int32), batch.astype(jnp.int32), tab)
    likelihood = _combine(lik_part)
    return likelihood, h_flat.reshape(N, G)
